# baseline (speedup 1.0000x reference)
import functools

import jax
import jax.numpy as jnp
from jax import lax
from jax.experimental import pallas as pl
from jax.experimental.pallas import tpu as pltpu
from jax.experimental.pallas import tpu_sc as plsc

N = 10000
E = 320000
C = 128
NP = 10240
NR = NP // 128
NC, NS, L = 2, 16, 16
NW = NC * NS
CH = 64
CPT = 160
GRP = 16
NGRP = CPT // GRP
NB = 4
EPT = CPT * CH
E_PAD = NW * EPT
DUMMY = N + 64

K1_EPT = E_PAD // NS
K1_SPAN = 2048
K1_OUT = K1_EPT // K1_SPAN

_MESH = plsc.VectorSubcoreMesh(core_axis_name="c", subcore_axis_name="s")


def _rsqrt16(x):
    i = lax.bitcast_convert_type(x, jnp.int32)
    i = jnp.int32(0x5F3759DF) - lax.shift_right_logical(i, 1)
    y = lax.bitcast_convert_type(i, jnp.float32)
    for _ in range(3):
        y = y * (1.5 - 0.5 * x * y * y)
    return y


_RED = NP // NS


@functools.partial(
    pl.kernel,
    out_type=jax.ShapeDtypeStruct((NP,), jnp.float32),
    mesh=_MESH,
    compiler_params=pltpu.CompilerParams(needs_layout_passes=False),
    scratch_types=[
        pltpu.VMEM((2, 2, K1_SPAN), jnp.int32),
        pltpu.SemaphoreType.DMA,
        pltpu.SemaphoreType.DMA,
        pltpu.VMEM((NP,), jnp.float32),
        pltpu.VMEM((_RED,), jnp.float32),
        pltpu.VMEM((_RED,), jnp.float32),
        pltpu.VMEM_SHARED((NS, NP), jnp.float32),
    ],
)
def _deg_dinv(pk1_hbm, dinv_hbm, epk, ksem0, ksem1, degl, redv, tmpv, degsp):
    cid = lax.axis_index("c")
    sid = lax.axis_index("s")
    ksems = (ksem0, ksem1)

    @pl.when(cid == 0)
    def _():
        zero16 = jnp.zeros((16,), jnp.float32)

        def _z(i, carry):
            degl[pl.ds(i * 16, 16)] = zero16
            return carry
        lax.fori_loop(0, NP // 16, _z, 0)

        pltpu.sync_copy(pk1_hbm.at[sid, 0], epk.at[0])
        pltpu.async_copy(pk1_hbm.at[sid, 1], epk.at[1], ksem1)

        def _span(o, carry):
            par = lax.bitwise_and(o, 1)

            for q in range(2):
                @pl.when((o > 0) & (par == q))
                def _():
                    pltpu.make_async_copy(
                        pk1_hbm.at[sid, o], epk.at[q], ksems[q]
                    ).wait()

            def _vec(i, c2):
                col16 = epk[par, 0, pl.ds(i * 16, 16)]
                ew16 = lax.bitcast_convert_type(
                    epk[par, 1, pl.ds(i * 16, 16)], jnp.float32
                )
                plsc.addupdate_scatter(degl, [col16], ew16)
                return c2
            lax.fori_loop(0, K1_SPAN // 16, _vec, 0)

            for q in range(2):
                @pl.when((o + 2 < K1_OUT) & (par == q))
                def _():
                    pltpu.async_copy(pk1_hbm.at[sid, o + 2], epk.at[q], ksems[q])
            return carry
        lax.fori_loop(0, K1_OUT, _span, 0)

        pltpu.sync_copy(degl, degsp.at[sid])
        plsc.subcore_barrier()

        pltpu.sync_copy(degsp.at[0, pl.ds(sid * _RED, _RED)], redv)
        for k in range(1, NS):
            pltpu.sync_copy(degsp.at[k, pl.ds(sid * _RED, _RED)], tmpv)

            def _acc(i, c2):
                sl = pl.ds(i * 16, 16)
                redv[sl] = redv[sl] + tmpv[sl]
                return c2
            lax.fori_loop(0, _RED // 16, _acc, 0)

        def _rs(i, c2):
            sl = pl.ds(i * 16, 16)
            redv[sl] = _rsqrt16(redv[sl] + 1.0)
            return c2
        lax.fori_loop(0, _RED // 16, _rs, 0)
        pltpu.sync_copy(redv, dinv_hbm.at[pl.ds(sid * _RED, _RED)])


@functools.partial(
    pl.kernel,
    out_type=jax.ShapeDtypeStruct((2, NP, 128), jnp.float32),
    mesh=_MESH,
    scratch_types=[
        pltpu.VMEM((2, GRP, 3, CH), jnp.int32),
        pltpu.VMEM((CH, 128), jnp.float32),
        pltpu.VMEM((CH, 128), jnp.float32),
        pltpu.VMEM((CH, 128), jnp.float32),
        pltpu.VMEM((CH, 128), jnp.float32),
        pltpu.SemaphoreType.DMA,
        pltpu.SemaphoreType.DMA,
        pltpu.SemaphoreType.DMA,
        pltpu.SemaphoreType.DMA,
        pltpu.VMEM_SHARED((NP, 128), jnp.float32),
    ],
)
def _aggregate(y_hbm, pk_hbm, acc_hbm,
               ibuf, buf0, buf1, buf2, buf3, sem0, sem1, sem2, sem3, accsp):
    cid = lax.axis_index("c")
    sid = lax.axis_index("s")
    w = cid * NS + sid
    bufs = (buf0, buf1, buf2, buf3)
    sems = (sem0, sem1, sem2, sem3)

    zero16 = jnp.zeros((16,), jnp.float32)

    def _z(i, carry):
        for k in range(8):
            buf0[i, pl.ds(16 * k, 16)] = zero16
        return carry
    lax.fori_loop(0, CH, _z, 0)
    rows_per_tile = NP // NS
    for t in range(rows_per_tile // CH):
        pltpu.sync_copy(buf0, accsp.at[pl.ds(sid * rows_per_tile + t * CH, CH)])
    plsc.subcore_barrier()

    pltpu.sync_copy(pk_hbm.at[w, 0], ibuf.at[0])
    for j in range(NB):
        pltpu.async_copy(y_hbm.at[ibuf.at[0, j, 0]], bufs[j], sems[j])

    def _outer(t, carry):
        for nb in range(NB):
            c = t * NB + nb
            g = lax.shift_right_logical(c, 4)
            p = lax.bitwise_and(g, 1)
            slot = lax.bitwise_and(c, GRP - 1)
            buf, sem = bufs[nb], sems[nb]
            pltpu.make_async_copy(y_hbm.at[ibuf.at[p, slot, 0]], buf, sem).wait()

            def _scale(gi, c2):
                sv = lax.bitcast_convert_type(
                    ibuf[p, slot, 2, pl.ds(gi * 16, 16)], jnp.float32
                )
                for j in range(16):
                    s = sv[j]
                    r = gi * 16 + j
                    for k in range(8):
                        sl = pl.ds(16 * k, 16)
                        buf[r, sl] = buf[r, sl] * s
                return c2
            lax.fori_loop(0, CH // 16, _scale, 0)

            pltpu.sync_copy(buf, accsp.at[ibuf.at[p, slot, 1]], add=True)

            @pl.when((slot == 0) & (g + 1 < NGRP))
            def _():
                pltpu.sync_copy(
                    pk_hbm.at[w, g + 1], ibuf.at[lax.bitwise_and(g + 1, 1)]
                )

            nxt = c + NB

            @pl.when(nxt < CPT)
            def _():
                g4 = lax.shift_right_logical(nxt, 4)
                p4 = lax.bitwise_and(g4, 1)
                s4 = lax.bitwise_and(nxt, GRP - 1)
                pltpu.async_copy(y_hbm.at[ibuf.at[p4, s4, 0]], buf, sem)
        return carry
    lax.fori_loop(0, CPT // NB, _outer, 0)
    plsc.subcore_barrier()

    for t in range(rows_per_tile // CH):
        base = sid * rows_per_tile + t * CH
        pltpu.sync_copy(accsp.at[pl.ds(base, CH)], acc_hbm.at[cid, pl.ds(base, CH)])


def _matmul_body(x_ref, w_ref, dv_ref, y_ref):
    y_ref[...] = (
        jnp.dot(x_ref[...], w_ref[...], preferred_element_type=jnp.float32)
        * dv_ref[...]
    )


def _final_body(a0_ref, a1_ref, y_ref, dv_ref, b_ref, o_ref):
    o_ref[...] = (a0_ref[...] + a1_ref[...] + y_ref[...]) * dv_ref[...] + b_ref[...]


_BM = 2048


def kernel(x, HE, HEW, W, b):
    row = HE[0].astype(jnp.int32)
    col = HE[1].astype(jnp.int32)
    pad = E_PAD - E
    row_p = jnp.concatenate([row, jnp.zeros((pad,), jnp.int32)])
    col_p = jnp.concatenate([col, jnp.full((pad,), DUMMY, jnp.int32)])
    ew_p = jnp.concatenate([HEW.astype(jnp.float32), jnp.zeros((pad,), jnp.float32)])
    x_pad = jnp.pad(x, ((0, NP - N), (0, 0)))

    ew_bits = lax.bitcast_convert_type(ew_p, jnp.int32)
    pk1 = jnp.stack(
        [
            col_p.reshape(NS, K1_OUT, K1_SPAN),
            ew_bits.reshape(NS, K1_OUT, K1_SPAN),
        ],
        axis=2,
    )
    dinv = _deg_dinv(pk1)
    dinv_col = dinv.reshape(NP, 1)

    y = pl.pallas_call(
        _matmul_body,
        grid=(NP // _BM,),
        in_specs=[
            pl.BlockSpec((_BM, C), lambda i: (i, 0)),
            pl.BlockSpec((C, C), lambda i: (0, 0)),
            pl.BlockSpec((_BM, 1), lambda i: (i, 0)),
        ],
        out_specs=pl.BlockSpec((_BM, C), lambda i: (i, 0)),
        out_shape=jax.ShapeDtypeStruct((NP, C), jnp.float32),
    )(x_pad, W, dinv_col)

    pk = jnp.stack(
        [
            row_p.reshape(NW, NGRP, GRP, CH),
            col_p.reshape(NW, NGRP, GRP, CH),
            ew_bits.reshape(NW, NGRP, GRP, CH),
        ],
        axis=3,
    )
    acc = _aggregate(y, pk)

    out = pl.pallas_call(
        _final_body,
        grid=(NP // _BM,),
        in_specs=[
            pl.BlockSpec((_BM, C), lambda i: (i, 0)),
            pl.BlockSpec((_BM, C), lambda i: (i, 0)),
            pl.BlockSpec((_BM, C), lambda i: (i, 0)),
            pl.BlockSpec((_BM, 1), lambda i: (i, 0)),
            pl.BlockSpec((1, C), lambda i: (0, 0)),
        ],
        out_specs=pl.BlockSpec((_BM, C), lambda i: (i, 0)),
        out_shape=jax.ShapeDtypeStruct((NP, C), jnp.float32),
    )(acc[0], acc[1], y, dinv_col, b.reshape(1, C))

    return out[:N]


# speedup vs baseline: 1.2334x; 1.2334x over previous
import functools

import jax
import jax.numpy as jnp
from jax import lax
from jax.experimental import pallas as pl
from jax.experimental.pallas import tpu as pltpu
from jax.experimental.pallas import tpu_sc as plsc

N = 10000
E = 320000
C = 128
NP = 10240
NR = NP // 128
NC, NS, L = 2, 16, 16
NW = NC * NS
CH = 64
CPT = 160
GRP = 16
NGRP = CPT // GRP
NB = 4
EPT = CPT * CH
E_PAD = NW * EPT
DUMMY = N + 64

K1_EPT = E_PAD // NS
K1_SPAN = 2048
K1_OUT = K1_EPT // K1_SPAN

_MESH = plsc.VectorSubcoreMesh(core_axis_name="c", subcore_axis_name="s")


def _rsqrt16(x):
    i = lax.bitcast_convert_type(x, jnp.int32)
    i = jnp.int32(0x5F3759DF) - lax.shift_right_logical(i, 1)
    y = lax.bitcast_convert_type(i, jnp.float32)
    for _ in range(3):
        y = y * (1.5 - 0.5 * x * y * y)
    return y


_RED = NP // NS


@functools.partial(
    pl.kernel,
    out_type=jax.ShapeDtypeStruct((NP,), jnp.float32),
    mesh=_MESH,
    compiler_params=pltpu.CompilerParams(needs_layout_passes=False),
    scratch_types=[
        pltpu.VMEM((K1_SPAN,), jnp.int32),
        pltpu.VMEM((K1_SPAN,), jnp.float32),
        pltpu.VMEM((NP,), jnp.float32),
        pltpu.VMEM((_RED,), jnp.float32),
        pltpu.VMEM((_RED,), jnp.float32),
        pltpu.VMEM_SHARED((NS, NP), jnp.float32),
    ],
)
def _deg_dinv(col_hbm, ew_hbm, dinv_hbm, colv, ewv, degl, redv, tmpv, degsp):
    cid = lax.axis_index("c")
    sid = lax.axis_index("s")

    @pl.when(cid == 0)
    def _():
        zero16 = jnp.zeros((16,), jnp.float32)

        def _z(i, carry):
            degl[pl.ds(i * 16, 16)] = zero16
            return carry
        lax.fori_loop(0, NP // 16, _z, 0)

        def _span(o, carry):
            base = sid * K1_EPT + o * K1_SPAN
            pltpu.sync_copy(col_hbm.at[pl.ds(base, K1_SPAN)], colv)
            pltpu.sync_copy(ew_hbm.at[pl.ds(base, K1_SPAN)], ewv)

            def _vec(i, c2):
                col16 = colv[pl.ds(i * 16, 16)]
                ew16 = ewv[pl.ds(i * 16, 16)]
                plsc.addupdate_scatter(degl, [col16], ew16)
                return c2
            lax.fori_loop(0, K1_SPAN // 16, _vec, 0)
            return carry
        lax.fori_loop(0, K1_OUT, _span, 0)

        pltpu.sync_copy(degl, degsp.at[sid])
        plsc.subcore_barrier()

        pltpu.sync_copy(degsp.at[0, pl.ds(sid * _RED, _RED)], redv)
        for k in range(1, NS):
            pltpu.sync_copy(degsp.at[k, pl.ds(sid * _RED, _RED)], tmpv)

            def _acc(i, c2):
                sl = pl.ds(i * 16, 16)
                redv[sl] = redv[sl] + tmpv[sl]
                return c2
            lax.fori_loop(0, _RED // 16, _acc, 0)

        def _rs(i, c2):
            sl = pl.ds(i * 16, 16)
            redv[sl] = _rsqrt16(redv[sl] + 1.0)
            return c2
        lax.fori_loop(0, _RED // 16, _rs, 0)
        pltpu.sync_copy(redv, dinv_hbm.at[pl.ds(sid * _RED, _RED)])


@functools.partial(
    pl.kernel,
    out_type=jax.ShapeDtypeStruct((2, NP, 128), jnp.float32),
    mesh=_MESH,
    scratch_types=[
        pltpu.VMEM((2, GRP, 3, CH), jnp.int32),
        pltpu.VMEM((CH, 128), jnp.float32),
        pltpu.VMEM((CH, 128), jnp.float32),
        pltpu.VMEM((CH, 128), jnp.float32),
        pltpu.VMEM((CH, 128), jnp.float32),
        pltpu.SemaphoreType.DMA,
        pltpu.SemaphoreType.DMA,
        pltpu.SemaphoreType.DMA,
        pltpu.SemaphoreType.DMA,
        pltpu.VMEM_SHARED((NP, 128), jnp.float32),
    ],
)
def _aggregate(y_hbm, pk_hbm, acc_hbm,
               ibuf, buf0, buf1, buf2, buf3, sem0, sem1, sem2, sem3, accsp):
    cid = lax.axis_index("c")
    sid = lax.axis_index("s")
    w = cid * NS + sid
    bufs = (buf0, buf1, buf2, buf3)
    sems = (sem0, sem1, sem2, sem3)

    zero16 = jnp.zeros((16,), jnp.float32)

    def _z(i, carry):
        for k in range(8):
            buf0[i, pl.ds(16 * k, 16)] = zero16
        return carry
    lax.fori_loop(0, CH, _z, 0)
    rows_per_tile = NP // NS
    for t in range(rows_per_tile // CH):
        pltpu.sync_copy(buf0, accsp.at[pl.ds(sid * rows_per_tile + t * CH, CH)])
    plsc.subcore_barrier()

    pltpu.sync_copy(pk_hbm.at[w, 0], ibuf.at[0])
    for j in range(NB):
        pltpu.async_copy(y_hbm.at[ibuf.at[0, j, 0]], bufs[j], sems[j])

    def _outer(t, carry):
        for nb in range(NB):
            c = t * NB + nb
            g = lax.shift_right_logical(c, 4)
            p = lax.bitwise_and(g, 1)
            slot = lax.bitwise_and(c, GRP - 1)
            buf, sem = bufs[nb], sems[nb]
            pltpu.make_async_copy(y_hbm.at[ibuf.at[p, slot, 0]], buf, sem).wait()

            def _scale(gi, c2):
                sv = lax.bitcast_convert_type(
                    ibuf[p, slot, 2, pl.ds(gi * 16, 16)], jnp.float32
                )
                for j in range(16):
                    s = sv[j]
                    r = gi * 16 + j
                    for k in range(8):
                        sl = pl.ds(16 * k, 16)
                        buf[r, sl] = buf[r, sl] * s
                return c2
            lax.fori_loop(0, CH // 16, _scale, 0)

            pltpu.sync_copy(buf, accsp.at[ibuf.at[p, slot, 1]], add=True)

            @pl.when((slot == 0) & (g + 1 < NGRP))
            def _():
                pltpu.sync_copy(
                    pk_hbm.at[w, g + 1], ibuf.at[lax.bitwise_and(g + 1, 1)]
                )

            nxt = c + NB

            @pl.when(nxt < CPT)
            def _():
                g4 = lax.shift_right_logical(nxt, 4)
                p4 = lax.bitwise_and(g4, 1)
                s4 = lax.bitwise_and(nxt, GRP - 1)
                pltpu.async_copy(y_hbm.at[ibuf.at[p4, s4, 0]], buf, sem)
        return carry
    lax.fori_loop(0, CPT // NB, _outer, 0)
    plsc.subcore_barrier()

    for t in range(rows_per_tile // CH):
        base = sid * rows_per_tile + t * CH
        pltpu.sync_copy(accsp.at[pl.ds(base, CH)], acc_hbm.at[cid, pl.ds(base, CH)])


def _matmul_body(x_ref, w_ref, dv_ref, y_ref):
    y_ref[...] = (
        jnp.dot(x_ref[...], w_ref[...], preferred_element_type=jnp.float32)
        * dv_ref[...]
    )


def _final_body(a0_ref, a1_ref, y_ref, dv_ref, b_ref, o_ref):
    o_ref[...] = (a0_ref[...] + a1_ref[...] + y_ref[...]) * dv_ref[...] + b_ref[...]


_BM = 2048


def kernel(x, HE, HEW, W, b):
    row = HE[0].astype(jnp.int32)
    col = HE[1].astype(jnp.int32)
    pad = E_PAD - E
    row_p = jnp.concatenate([row, jnp.zeros((pad,), jnp.int32)])
    col_p = jnp.concatenate([col, jnp.full((pad,), DUMMY, jnp.int32)])
    ew_p = jnp.concatenate([HEW.astype(jnp.float32), jnp.zeros((pad,), jnp.float32)])
    x_pad = jnp.pad(x, ((0, NP - N), (0, 0)))

    dinv = _deg_dinv(col_p, ew_p)
    dinv_col = dinv.reshape(NP, 1)

    y = pl.pallas_call(
        _matmul_body,
        grid=(NP // _BM,),
        in_specs=[
            pl.BlockSpec((_BM, C), lambda i: (i, 0)),
            pl.BlockSpec((C, C), lambda i: (0, 0)),
            pl.BlockSpec((_BM, 1), lambda i: (i, 0)),
        ],
        out_specs=pl.BlockSpec((_BM, C), lambda i: (i, 0)),
        out_shape=jax.ShapeDtypeStruct((NP, C), jnp.float32),
    )(x_pad, W, dinv_col)

    pk = jnp.stack(
        [
            row_p.reshape(NW, NGRP, GRP, CH),
            col_p.reshape(NW, NGRP, GRP, CH),
            lax.bitcast_convert_type(ew_p, jnp.int32).reshape(NW, NGRP, GRP, CH),
        ],
        axis=3,
    )
    acc = _aggregate(y, pk)

    out = pl.pallas_call(
        _final_body,
        grid=(NP // _BM,),
        in_specs=[
            pl.BlockSpec((_BM, C), lambda i: (i, 0)),
            pl.BlockSpec((_BM, C), lambda i: (i, 0)),
            pl.BlockSpec((_BM, C), lambda i: (i, 0)),
            pl.BlockSpec((_BM, 1), lambda i: (i, 0)),
            pl.BlockSpec((1, C), lambda i: (0, 0)),
        ],
        out_specs=pl.BlockSpec((_BM, C), lambda i: (i, 0)),
        out_shape=jax.ShapeDtypeStruct((NP, C), jnp.float32),
    )(acc[0], acc[1], y, dinv_col, b.reshape(1, C))

    return out[:N]
